# baseline (speedup 1.0000x reference)
import functools

import jax
import jax.numpy as jnp
from jax import lax
from jax.experimental import pallas as pl
from jax.experimental.pallas import tpu as pltpu
from jax.experimental.pallas import tpu_sc as plsc

SEQ = 200
DIM = 64
BATCH = 4096
N_ROWS = BATCH * SEQ
NUM_CORES = 2
NUM_SUBCORES = 16
NW = NUM_CORES * NUM_SUBCORES
SEQ_PER_W = N_ROWS // (SEQ * NW)
IDX_MINOR = 40
IDX_PER_CHUNK = SEQ // IDX_MINOR
IDX_ROWS_PER_W = SEQ_PER_W * IDX_PER_CHUNK
NBUF = 4
LANES = 16
SCALE = 8.0


def _body(inp_ref, tok_ref, pos_ref, out_ref,
          idx_v, pos_v, rows0, rows1, rows2, rows3,
          gsem0, gsem1, gsem2, gsem3, ssem0, ssem1, ssem2, ssem3):
  rows = (rows0, rows1, rows2, rows3)
  gsem = (gsem0, gsem1, gsem2, gsem3)
  ssem = (ssem0, ssem1, ssem2, ssem3)

  w = lax.axis_index("s") * NUM_CORES + lax.axis_index("c")
  base_seq = w * SEQ_PER_W

  def start_gather(b, c):
    for g in range(IDX_PER_CHUNK):
      pltpu.async_copy(
          tok_ref.at[idx_v.at[c * IDX_PER_CHUNK + g]],
          rows[b].at[pl.ds(g * IDX_MINOR, IDX_MINOR)],
          gsem[b])

  def wait_gather(b):
    pltpu.make_async_copy(tok_ref.at[pl.ds(0, SEQ)], rows[b], gsem[b]).wait()

  def start_scatter(b, c):
    pltpu.async_copy(rows[b],
                     out_ref.at[pl.ds((base_seq + c) * SEQ, SEQ)],
                     ssem[b])

  def wait_scatter(b):
    pltpu.make_async_copy(rows[b], out_ref.at[pl.ds(0, SEQ)], ssem[b]).wait()

  def compute(b):
    @pl.loop(0, SEQ)
    def _(i):
      for q in range(DIM // LANES):
        sl = pl.ds(q * LANES, LANES)
        rows[b][i, sl] = rows[b][i, sl] * SCALE + pos_v[i, sl]

  pltpu.sync_copy(inp_ref.at[pl.ds(w * IDX_ROWS_PER_W, IDX_ROWS_PER_W)], idx_v)
  pltpu.sync_copy(pos_ref, pos_v)

  for b in range(NBUF):
    start_gather(b, jnp.int32(b))

  @pl.loop(0, SEQ_PER_W // NBUF)
  def _(grp):
    for b in range(NBUF):
      c = grp * NBUF + b
      wait_gather(b)
      compute(b)
      start_scatter(b, c)
      nxt = c + NBUF

      @pl.when(nxt < SEQ_PER_W)
      def _():
        wait_scatter(b)
        start_gather(b, nxt)

  for b in range(NBUF):
    wait_scatter(b)


@jax.jit
def _embed(inputs2d, token_table, pos_table):
  mesh = plsc.VectorSubcoreMesh(core_axis_name="c", subcore_axis_name="s")
  run = pl.kernel(
      _body,
      out_type=jax.ShapeDtypeStruct((N_ROWS, DIM), jnp.float32),
      mesh=mesh,
      compiler_params=pltpu.CompilerParams(use_tc_tiling_on_sc=False),
      scratch_types=[
          pltpu.VMEM((NW * IDX_ROWS_PER_W // NW, IDX_MINOR), jnp.int32),
          pltpu.VMEM((SEQ, DIM), jnp.float32),
      ] + [pltpu.VMEM((SEQ, DIM), jnp.float32) for _ in range(NBUF)]
        + [pltpu.SemaphoreType.DMA for _ in range(2 * NBUF)],
  )
  return run(inputs2d, token_table, pos_table)


def kernel(inputs, token_table, pos_table):
  inputs2d = inputs.reshape(NW * IDX_ROWS_PER_W, IDX_MINOR).astype(jnp.int32)
  out = _embed(inputs2d, token_table, pos_table)
  return out.reshape(BATCH, SEQ, DIM)
